# baseline (speedup 1.0000x reference)
import jax
import jax.numpy as jnp
from jax import lax
from jax.experimental import pallas as pl
from jax.experimental.pallas import tpu as pltpu
from jax.experimental.pallas import tpu_sc as plsc

N = 20000
G = 128
C = 80
NP = 20480
TB = 2048
NW = 32
RPW = 640
CHUNKS = RPW // 16


def _tc_body(reg_ref, tgt_ref, cls_ref,
             cx_ref, cy_ref, sidx_ref, gcx_ref, gcy_ref, glb_ref):
    regt = jnp.transpose(reg_ref[...])
    clst = jnp.transpose(cls_ref[...])
    tgtt = jnp.transpose(tgt_ref[...])

    cx_ref[...] = (regt[0] + regt[2]) / 2.0
    cy_ref[...] = (regt[1] + regt[3]) / 2.0

    maxv = jnp.max(clst, axis=0)
    ciota = lax.broadcasted_iota(jnp.int32, clst.shape, 0)
    cidx = jnp.min(jnp.where(clst == maxv[None, :], ciota, C), axis=0)
    sidx_ref[...] = jnp.where(maxv > 0.0, cidx, -1)

    gcx_ref[...] = (tgtt[0] + tgtt[2]) / 2.0
    gcy_ref[...] = (tgtt[1] + tgtt[3]) / 2.0
    glb_ref[...] = tgtt[4]


def _sc_body(cx_hbm, cy_hbm, sidx_hbm, gcx_hbm, gcy_hbm, glb_hbm,
             asg_hbm, dis_hbm, lbl_hbm,
             cx_v, cy_v, sidx_v, gcx_v, gcy_v, glb_v,
             asg_v, dis_v, lbl_v, sem):
    wid = lax.axis_index("s") * 2 + lax.axis_index("c")
    base = jnp.minimum(wid * RPW, N - RPW)

    iota = jnp.arange(16, dtype=jnp.int32)

    cps = [
        pltpu.async_copy(cx_hbm.at[pl.ds(base, RPW)], cx_v, sem),
        pltpu.async_copy(cy_hbm.at[pl.ds(base, RPW)], cy_v, sem),
        pltpu.async_copy(sidx_hbm.at[pl.ds(base, RPW)], sidx_v, sem),
        pltpu.async_copy(gcx_hbm, gcx_v, sem),
        pltpu.async_copy(gcy_hbm, gcy_v, sem),
        pltpu.async_copy(glb_hbm, glb_v, sem),
    ]
    for cp in cps:
        cp.wait()

    gcx_ch = [gcx_v[pl.ds(16 * k, 16)] for k in range(G // 16)]
    gcy_ch = [gcy_v[pl.ds(16 * k, 16)] for k in range(G // 16)]

    NH = 4
    NS = 2
    GB = G // NS

    def chunk(j, carry):
        rows_h, cx_h, cy_h = [], [], []
        for h in range(NH):
            rows = iota + (j * (16 * NH) + 16 * h)
            rows_h.append(rows)
            cx_h.append(plsc.load_gather(cx_v, [rows]))
            cy_h.append(plsc.load_gather(cy_v, [rows]))

        inf16 = jnp.full((16,), jnp.inf, jnp.float32)
        zero16 = jnp.zeros((16,), jnp.int32)
        best = [[inf16 for _ in range(NS)] for _ in range(NH)]
        bidx = [[zero16 for _ in range(NS)] for _ in range(NH)]
        for s in range(NS):
            for gi in range(GB):
                g = s * GB + gi
                gx = gcx_ch[g // 16][g % 16]
                gy = gcy_ch[g // 16][g % 16]
                for h in range(NH):
                    dx = cx_h[h] - gx
                    dy = cy_h[h] - gy
                    d2 = dx * dx + dy * dy
                    m = d2 < best[h][s]
                    best[h][s] = jnp.where(m, d2, best[h][s])
                    bidx[h][s] = jnp.where(m, jnp.int32(g), bidx[h][s])

        for h in range(NH):
            b, bi = best[h][0], bidx[h][0]
            for s in range(1, NS):
                m = best[h][s] < b
                b = jnp.where(m, best[h][s], b)
                bi = jnp.where(m, bidx[h][s], bi)

            glab = plsc.load_gather(glb_v, [bi])
            glab_i = glab.astype(jnp.int32)

            i = lax.bitcast_convert_type(b, jnp.int32)
            i = jnp.int32(0x1FBD1DF5) + lax.shift_right_arithmetic(i, 1)
            y = lax.bitcast_convert_type(i, jnp.float32)
            y = 0.5 * (y + b / y)
            y = 0.5 * (y + b / y)
            y = 0.5 * (y + b / y)

            si = plsc.load_gather(sidx_v, [rows_h[h]])
            pos = si == glab_i
            asg = jnp.where(pos, bi + 1, 0)
            albl = jnp.where(pos, glab_i, jnp.int32(-1))

            plsc.store_scatter(asg_v, [rows_h[h]], asg)
            plsc.store_scatter(dis_v, [rows_h[h]], y)
            plsc.store_scatter(lbl_v, [rows_h[h]], albl)
        return carry

    lax.fori_loop(0, CHUNKS // NH, chunk, 0)

    pltpu.sync_copy(asg_v, asg_hbm.at[pl.ds(base, RPW)])
    pltpu.sync_copy(dis_v, dis_hbm.at[pl.ds(base, RPW)])
    pltpu.sync_copy(lbl_v, lbl_hbm.at[pl.ds(base, RPW)])


@jax.jit
def _run(reg_pred, targets, cls_pred):
    nb = NP // TB
    cx, cy, sidx, gcx, gcy, glb = pl.pallas_call(
        _tc_body,
        grid=(nb,),
        in_specs=[
            pl.BlockSpec((TB, 4), lambda i: (i, 0)),
            pl.BlockSpec((G, 5), lambda i: (0, 0)),
            pl.BlockSpec((TB, C), lambda i: (i, 0)),
        ],
        out_specs=(
            pl.BlockSpec((TB,), lambda i: (i,)),
            pl.BlockSpec((TB,), lambda i: (i,)),
            pl.BlockSpec((TB,), lambda i: (i,)),
            pl.BlockSpec((G,), lambda i: (0,)),
            pl.BlockSpec((G,), lambda i: (0,)),
            pl.BlockSpec((G,), lambda i: (0,)),
        ),
        out_shape=(
            jax.ShapeDtypeStruct((NP,), jnp.float32),
            jax.ShapeDtypeStruct((NP,), jnp.float32),
            jax.ShapeDtypeStruct((NP,), jnp.int32),
            jax.ShapeDtypeStruct((G,), jnp.float32),
            jax.ShapeDtypeStruct((G,), jnp.float32),
            jax.ShapeDtypeStruct((G,), jnp.float32),
        ),
    )(reg_pred, targets, cls_pred)

    mesh = plsc.VectorSubcoreMesh(core_axis_name="c", subcore_axis_name="s")
    f = pl.kernel(
        _sc_body,
        mesh=mesh,
        compiler_params=pltpu.CompilerParams(needs_layout_passes=False),
        out_type=(
            jax.ShapeDtypeStruct((N,), jnp.int32),
            jax.ShapeDtypeStruct((N,), jnp.float32),
            jax.ShapeDtypeStruct((N,), jnp.int32),
        ),
        scratch_types=[
            pltpu.VMEM((RPW,), jnp.float32),
            pltpu.VMEM((RPW,), jnp.float32),
            pltpu.VMEM((RPW,), jnp.int32),
            pltpu.VMEM((G,), jnp.float32),
            pltpu.VMEM((G,), jnp.float32),
            pltpu.VMEM((G,), jnp.float32),
            pltpu.VMEM((RPW,), jnp.int32),
            pltpu.VMEM((RPW,), jnp.float32),
            pltpu.VMEM((RPW,), jnp.int32),
            pltpu.SemaphoreType.DMA,
        ],
    )
    return f(cx, cy, sidx, gcx, gcy, glb)


def kernel(reg_pred, targets, num_level_bboxes, cls_pred):
    asg, dis, lbl = _run(reg_pred, targets, cls_pred)
    return (asg, dis, lbl, reg_pred, targets)


# speedup vs baseline: 1.1891x; 1.1891x over previous
import jax
import jax.numpy as jnp
from jax import lax
from jax.experimental import pallas as pl
from jax.experimental.pallas import tpu as pltpu
from jax.experimental.pallas import tpu_sc as plsc

N = 20000
G = 128
C = 80
NP = 20480
TB = 2048
NW = 32
RPW = 640
CHUNKS = RPW // 16


def _tc1_body(reg_ref, tgt_ref, cx_ref, cy_ref, gcx_ref, gcy_ref, glb_ref):
    regt = jnp.transpose(reg_ref[...])
    tgtt = jnp.transpose(tgt_ref[...])
    cx_ref[...] = (regt[0] + regt[2]) / 2.0
    cy_ref[...] = (regt[1] + regt[3]) / 2.0
    gcx_ref[...] = (tgtt[0] + tgtt[2]) / 2.0
    gcy_ref[...] = (tgtt[1] + tgtt[3]) / 2.0
    glb_ref[...] = tgtt[4]


def _tc2_body(cls_ref, sidx_ref):
    clst = jnp.transpose(cls_ref[...])
    maxv = jnp.max(clst, axis=0)
    ciota = lax.broadcasted_iota(jnp.int32, clst.shape, 0)
    cidx = jnp.min(jnp.where(clst == maxv[None, :], ciota, C), axis=0)
    sidx_ref[...] = jnp.where(maxv > 0.0, cidx, -1)


def _tc3_body(bidx_ref, d2_ref, glb_ref, sidx_ref, asg_ref, dis_ref, lbl_ref):
    bi = bidx_ref[...]
    glab_i = glb_ref[...]
    pos = sidx_ref[...] == glab_i
    asg_ref[...] = jnp.where(pos, bi + 1, 0)
    dis_ref[...] = jnp.sqrt(d2_ref[...])
    lbl_ref[...] = jnp.where(pos, glab_i, -1)


def _sc_body(cx_hbm, cy_hbm, gcx_hbm, gcy_hbm, glb_hbm,
             bidx_hbm, d2_hbm, glbo_hbm,
             cx_v, cy_v, gcx_v, gcy_v, glb_v,
             bidx_v, d2_v, glbo_v, sem):
    wid = lax.axis_index("s") * 2 + lax.axis_index("c")
    base = jnp.minimum(wid * RPW, N - RPW)

    iota = jnp.arange(16, dtype=jnp.int32)

    cps = [
        pltpu.async_copy(cx_hbm.at[pl.ds(base, RPW)], cx_v, sem),
        pltpu.async_copy(cy_hbm.at[pl.ds(base, RPW)], cy_v, sem),
        pltpu.async_copy(gcx_hbm, gcx_v, sem),
        pltpu.async_copy(gcy_hbm, gcy_v, sem),
        pltpu.async_copy(glb_hbm, glb_v, sem),
    ]
    for cp in cps:
        cp.wait()

    gcx_ch = [gcx_v[pl.ds(16 * k, 16)] for k in range(G // 16)]
    gcy_ch = [gcy_v[pl.ds(16 * k, 16)] for k in range(G // 16)]

    NH = 2
    NS = 4
    GB = G // NS

    def chunk(j, carry):
        rows_h, cx_h, cy_h = [], [], []
        for h in range(NH):
            rows = iota + (j * (16 * NH) + 16 * h)
            rows_h.append(rows)
            cx_h.append(plsc.load_gather(cx_v, [rows]))
            cy_h.append(plsc.load_gather(cy_v, [rows]))

        inf16 = jnp.full((16,), jnp.inf, jnp.float32)
        zero16 = jnp.zeros((16,), jnp.int32)
        best = [[inf16 for _ in range(NS)] for _ in range(NH)]
        bidx = [[zero16 for _ in range(NS)] for _ in range(NH)]
        for s in range(NS):
            for gi in range(GB):
                g = s * GB + gi
                gx = gcx_ch[g // 16][g % 16]
                gy = gcy_ch[g // 16][g % 16]
                for h in range(NH):
                    dx = cx_h[h] - gx
                    dy = cy_h[h] - gy
                    d2 = dx * dx + dy * dy
                    m = d2 < best[h][s]
                    best[h][s] = jnp.where(m, d2, best[h][s])
                    bidx[h][s] = jnp.where(m, jnp.int32(g), bidx[h][s])

        for h in range(NH):
            b, bi = best[h][0], bidx[h][0]
            for s in range(1, NS):
                m = best[h][s] < b
                b = jnp.where(m, best[h][s], b)
                bi = jnp.where(m, bidx[h][s], bi)

            glab = plsc.load_gather(glb_v, [bi])

            plsc.store_scatter(bidx_v, [rows_h[h]], bi)
            plsc.store_scatter(d2_v, [rows_h[h]], b)
            plsc.store_scatter(glbo_v, [rows_h[h]], glab.astype(jnp.int32))
        return carry

    lax.fori_loop(0, CHUNKS // NH, chunk, 0)

    pltpu.sync_copy(bidx_v, bidx_hbm.at[pl.ds(base, RPW)])
    pltpu.sync_copy(d2_v, d2_hbm.at[pl.ds(base, RPW)])
    pltpu.sync_copy(glbo_v, glbo_hbm.at[pl.ds(base, RPW)])


@jax.jit
def _run(reg_pred, targets, cls_pred):
    nb = NP // TB
    cx, cy, gcx, gcy, glb = pl.pallas_call(
        _tc1_body,
        grid=(nb,),
        in_specs=[
            pl.BlockSpec((TB, 4), lambda i: (i, 0)),
            pl.BlockSpec((G, 5), lambda i: (0, 0)),
        ],
        out_specs=(
            pl.BlockSpec((TB,), lambda i: (i,)),
            pl.BlockSpec((TB,), lambda i: (i,)),
            pl.BlockSpec((G,), lambda i: (0,)),
            pl.BlockSpec((G,), lambda i: (0,)),
            pl.BlockSpec((G,), lambda i: (0,)),
        ),
        out_shape=(
            jax.ShapeDtypeStruct((NP,), jnp.float32),
            jax.ShapeDtypeStruct((NP,), jnp.float32),
            jax.ShapeDtypeStruct((G,), jnp.float32),
            jax.ShapeDtypeStruct((G,), jnp.float32),
            jax.ShapeDtypeStruct((G,), jnp.float32),
        ),
    )(reg_pred, targets)

    mesh = plsc.VectorSubcoreMesh(core_axis_name="c", subcore_axis_name="s")
    sc = pl.kernel(
        _sc_body,
        mesh=mesh,
        compiler_params=pltpu.CompilerParams(needs_layout_passes=False),
        out_type=(
            jax.ShapeDtypeStruct((N,), jnp.int32),
            jax.ShapeDtypeStruct((N,), jnp.float32),
            jax.ShapeDtypeStruct((N,), jnp.int32),
        ),
        scratch_types=[
            pltpu.VMEM((RPW,), jnp.float32),
            pltpu.VMEM((RPW,), jnp.float32),
            pltpu.VMEM((G,), jnp.float32),
            pltpu.VMEM((G,), jnp.float32),
            pltpu.VMEM((G,), jnp.float32),
            pltpu.VMEM((RPW,), jnp.int32),
            pltpu.VMEM((RPW,), jnp.float32),
            pltpu.VMEM((RPW,), jnp.int32),
            pltpu.SemaphoreType.DMA,
        ],
    )
    bidx, d2m, glbi = sc(cx, cy, gcx, gcy, glb)

    sidx = pl.pallas_call(
        _tc2_body,
        grid=(nb,),
        in_specs=[pl.BlockSpec((TB, C), lambda i: (i, 0))],
        out_specs=pl.BlockSpec((TB,), lambda i: (i,)),
        out_shape=jax.ShapeDtypeStruct((NP,), jnp.int32),
    )(cls_pred)

    asg, dis, lbl = pl.pallas_call(
        _tc3_body,
        out_shape=(
            jax.ShapeDtypeStruct((N,), jnp.int32),
            jax.ShapeDtypeStruct((N,), jnp.float32),
            jax.ShapeDtypeStruct((N,), jnp.int32),
        ),
    )(bidx, d2m, glbi, sidx[:N])
    return asg, dis, lbl


def kernel(reg_pred, targets, num_level_bboxes, cls_pred):
    asg, dis, lbl = _run(reg_pred, targets, cls_pred)
    return (asg, dis, lbl, reg_pred, targets)


# speedup vs baseline: 1.2893x; 1.0842x over previous
import jax
import jax.numpy as jnp
from jax import lax
from jax.experimental import pallas as pl
from jax.experimental.pallas import tpu as pltpu
from jax.experimental.pallas import tpu_sc as plsc

N = 20000
G = 128
C = 80
NP = 20480
TB = 2048
NW = 32
RPW = 640
CHUNKS = RPW // 16


def _tc1_body(rx0_ref, ry0_ref, rx1_ref, ry1_ref, tgt_ref,
              cx_ref, cy_ref, gcx_ref, gcy_ref, glb_ref):
    tgtt = jnp.transpose(tgt_ref[...])
    cx_ref[...] = (rx0_ref[...] + rx1_ref[...]) / 2.0
    cy_ref[...] = (ry0_ref[...] + ry1_ref[...]) / 2.0
    gcx_ref[...] = (tgtt[0] + tgtt[2]) / 2.0
    gcy_ref[...] = (tgtt[1] + tgtt[3]) / 2.0
    glb_ref[...] = tgtt[4]


def _tc2_body(cls_ref, sidx_ref):
    clst = jnp.transpose(cls_ref[...])
    maxv = jnp.max(clst, axis=0)
    ciota = lax.broadcasted_iota(jnp.int32, clst.shape, 0)
    cidx = jnp.min(jnp.where(clst == maxv[None, :], ciota, C), axis=0)
    sidx_ref[...] = jnp.where(maxv > 0.0, cidx, -1)


def _tc3_body(bidx_ref, d2_ref, glb_ref, sidx_ref, asg_ref, dis_ref, lbl_ref):
    bi = bidx_ref[...]
    glab_i = glb_ref[...]
    pos = sidx_ref[...] == glab_i
    asg_ref[...] = jnp.where(pos, bi + 1, 0)
    dis_ref[...] = jnp.sqrt(d2_ref[...])
    lbl_ref[...] = jnp.where(pos, glab_i, -1)


def _sc_body(cx_hbm, cy_hbm, gcx_hbm, gcy_hbm, glb_hbm,
             bidx_hbm, d2_hbm, glbo_hbm,
             cx_v, cy_v, gcx_v, gcy_v, glb_v,
             bidx_v, d2_v, glbo_v, sem):
    wid = lax.axis_index("s") * 2 + lax.axis_index("c")
    base = jnp.minimum(wid * RPW, N - RPW)

    iota = jnp.arange(16, dtype=jnp.int32)

    cps = [
        pltpu.async_copy(cx_hbm.at[pl.ds(base, RPW)], cx_v, sem),
        pltpu.async_copy(cy_hbm.at[pl.ds(base, RPW)], cy_v, sem),
        pltpu.async_copy(gcx_hbm, gcx_v, sem),
        pltpu.async_copy(gcy_hbm, gcy_v, sem),
        pltpu.async_copy(glb_hbm, glb_v, sem),
    ]
    for cp in cps:
        cp.wait()

    gcx_ch = [gcx_v[pl.ds(16 * k, 16)] for k in range(G // 16)]
    gcy_ch = [gcy_v[pl.ds(16 * k, 16)] for k in range(G // 16)]

    NH = 2
    NS = 4
    GB = G // NS

    def chunk(j, carry):
        rows_h, cx_h, cy_h = [], [], []
        for h in range(NH):
            rows = iota + (j * (16 * NH) + 16 * h)
            rows_h.append(rows)
            cx_h.append(plsc.load_gather(cx_v, [rows]))
            cy_h.append(plsc.load_gather(cy_v, [rows]))

        inf16 = jnp.full((16,), jnp.inf, jnp.float32)
        zero16 = jnp.zeros((16,), jnp.int32)
        best = [[inf16 for _ in range(NS)] for _ in range(NH)]
        bidx = [[zero16 for _ in range(NS)] for _ in range(NH)]
        for s in range(NS):
            for gi in range(GB):
                g = s * GB + gi
                gx = gcx_ch[g // 16][g % 16]
                gy = gcy_ch[g // 16][g % 16]
                for h in range(NH):
                    dx = cx_h[h] - gx
                    dy = cy_h[h] - gy
                    d2 = dx * dx + dy * dy
                    m = d2 < best[h][s]
                    best[h][s] = jnp.where(m, d2, best[h][s])
                    bidx[h][s] = jnp.where(m, jnp.int32(g), bidx[h][s])

        for h in range(NH):
            b, bi = best[h][0], bidx[h][0]
            for s in range(1, NS):
                m = best[h][s] < b
                b = jnp.where(m, best[h][s], b)
                bi = jnp.where(m, bidx[h][s], bi)

            glab = plsc.load_gather(glb_v, [bi])

            plsc.store_scatter(bidx_v, [rows_h[h]], bi)
            plsc.store_scatter(d2_v, [rows_h[h]], b)
            plsc.store_scatter(glbo_v, [rows_h[h]], glab.astype(jnp.int32))
        return carry

    lax.fori_loop(0, CHUNKS // NH, chunk, 0)

    pltpu.sync_copy(bidx_v, bidx_hbm.at[pl.ds(base, RPW)])
    pltpu.sync_copy(d2_v, d2_hbm.at[pl.ds(base, RPW)])
    pltpu.sync_copy(glbo_v, glbo_hbm.at[pl.ds(base, RPW)])


@jax.jit
def _run(reg_pred, targets, cls_pred):
    nb = NP // TB
    pad = NP - N
    rx0 = jnp.pad(reg_pred[:, 0], (0, pad))
    ry0 = jnp.pad(reg_pred[:, 1], (0, pad))
    rx1 = jnp.pad(reg_pred[:, 2], (0, pad))
    ry1 = jnp.pad(reg_pred[:, 3], (0, pad))
    cx, cy, gcx, gcy, glb = pl.pallas_call(
        _tc1_body,
        grid=(nb,),
        in_specs=[
            pl.BlockSpec((TB,), lambda i: (i,)),
            pl.BlockSpec((TB,), lambda i: (i,)),
            pl.BlockSpec((TB,), lambda i: (i,)),
            pl.BlockSpec((TB,), lambda i: (i,)),
            pl.BlockSpec((G, 5), lambda i: (0, 0)),
        ],
        out_specs=(
            pl.BlockSpec((TB,), lambda i: (i,)),
            pl.BlockSpec((TB,), lambda i: (i,)),
            pl.BlockSpec((G,), lambda i: (0,)),
            pl.BlockSpec((G,), lambda i: (0,)),
            pl.BlockSpec((G,), lambda i: (0,)),
        ),
        out_shape=(
            jax.ShapeDtypeStruct((NP,), jnp.float32),
            jax.ShapeDtypeStruct((NP,), jnp.float32),
            jax.ShapeDtypeStruct((G,), jnp.float32),
            jax.ShapeDtypeStruct((G,), jnp.float32),
            jax.ShapeDtypeStruct((G,), jnp.float32),
        ),
    )(rx0, ry0, rx1, ry1, targets)

    mesh = plsc.VectorSubcoreMesh(core_axis_name="c", subcore_axis_name="s")
    sc = pl.kernel(
        _sc_body,
        mesh=mesh,
        compiler_params=pltpu.CompilerParams(needs_layout_passes=False),
        out_type=(
            jax.ShapeDtypeStruct((N,), jnp.int32),
            jax.ShapeDtypeStruct((N,), jnp.float32),
            jax.ShapeDtypeStruct((N,), jnp.int32),
        ),
        scratch_types=[
            pltpu.VMEM((RPW,), jnp.float32),
            pltpu.VMEM((RPW,), jnp.float32),
            pltpu.VMEM((G,), jnp.float32),
            pltpu.VMEM((G,), jnp.float32),
            pltpu.VMEM((G,), jnp.float32),
            pltpu.VMEM((RPW,), jnp.int32),
            pltpu.VMEM((RPW,), jnp.float32),
            pltpu.VMEM((RPW,), jnp.int32),
            pltpu.SemaphoreType.DMA,
        ],
    )
    bidx, d2m, glbi = sc(cx, cy, gcx, gcy, glb)

    sidx = pl.pallas_call(
        _tc2_body,
        grid=(nb,),
        in_specs=[pl.BlockSpec((TB, C), lambda i: (i, 0))],
        out_specs=pl.BlockSpec((TB,), lambda i: (i,)),
        out_shape=jax.ShapeDtypeStruct((NP,), jnp.int32),
    )(cls_pred)

    asg, dis, lbl = pl.pallas_call(
        _tc3_body,
        out_shape=(
            jax.ShapeDtypeStruct((N,), jnp.int32),
            jax.ShapeDtypeStruct((N,), jnp.float32),
            jax.ShapeDtypeStruct((N,), jnp.int32),
        ),
    )(bidx, d2m, glbi, sidx[:N])
    return asg, dis, lbl


def kernel(reg_pred, targets, num_level_bboxes, cls_pred):
    asg, dis, lbl = _run(reg_pred, targets, cls_pred)
    return (asg, dis, lbl, reg_pred, targets)


# speedup vs baseline: 1.3222x; 1.0255x over previous
import jax
import jax.numpy as jnp
from jax import lax
from jax.experimental import pallas as pl
from jax.experimental.pallas import tpu as pltpu
from jax.experimental.pallas import tpu_sc as plsc

N = 20000
G = 128
C = 80
NP = 20480
TB = 2048
NW = 32
RPW = 640
CHUNKS = RPW // 16


def _tc1_body(rx0_ref, ry0_ref, rx1_ref, ry1_ref, tgt_ref,
              cx_ref, cy_ref, gcx_ref, gcy_ref, glb_ref):
    tgtt = jnp.transpose(tgt_ref[...])
    cx_ref[...] = (rx0_ref[...] + rx1_ref[...]) / 2.0
    cy_ref[...] = (ry0_ref[...] + ry1_ref[...]) / 2.0
    gcx_ref[...] = (tgtt[0] + tgtt[2]) / 2.0
    gcy_ref[...] = (tgtt[1] + tgtt[3]) / 2.0
    glb_ref[...] = tgtt[4]


def _tc2_body(cls_ref, sidx_ref):
    clst = jnp.transpose(cls_ref[...])
    maxv = jnp.max(clst, axis=0)
    ciota = lax.broadcasted_iota(jnp.int32, clst.shape, 0)
    cidx = jnp.min(jnp.where(clst == maxv[None, :], ciota, C), axis=0)
    sidx_ref[...] = jnp.where(maxv > 0.0, cidx, -1)


def _tc3_body(bidx_ref, d2_ref, glb_ref, sidx_ref, asg_ref, dis_ref, lbl_ref):
    bi = bidx_ref[...]
    glab_i = glb_ref[...]
    pos = sidx_ref[...] == glab_i
    asg_ref[...] = jnp.where(pos, bi + 1, 0)
    dis_ref[...] = jnp.sqrt(d2_ref[...])
    lbl_ref[...] = jnp.where(pos, glab_i, -1)


def _sc_body(cx_hbm, cy_hbm, gcx_hbm, gcy_hbm, glb_hbm,
             bidx_hbm, d2_hbm, glbo_hbm,
             cx_v, cy_v, gcx_v, gcy_v, glb_v,
             bidx_v, d2_v, glbo_v, sem):
    wid = lax.axis_index("s") * 2 + lax.axis_index("c")
    base = jnp.minimum(wid * RPW, N - RPW)

    iota = jnp.arange(16, dtype=jnp.int32)

    cps = [
        pltpu.async_copy(cx_hbm.at[pl.ds(base, RPW)], cx_v, sem),
        pltpu.async_copy(cy_hbm.at[pl.ds(base, RPW)], cy_v, sem),
        pltpu.async_copy(gcx_hbm, gcx_v, sem),
        pltpu.async_copy(gcy_hbm, gcy_v, sem),
        pltpu.async_copy(glb_hbm, glb_v, sem),
    ]
    for cp in cps:
        cp.wait()

    gcx_ch = [gcx_v[pl.ds(16 * k, 16)] for k in range(G // 16)]
    gcy_ch = [gcy_v[pl.ds(16 * k, 16)] for k in range(G // 16)]

    NH = 2
    NS = 4
    GB = G // NS

    def chunk(j, carry):
        rows_h, cx_h, cy_h = [], [], []
        for h in range(NH):
            rows = iota + (j * (16 * NH) + 16 * h)
            rows_h.append(rows)
            cx_h.append(plsc.load_gather(cx_v, [rows]))
            cy_h.append(plsc.load_gather(cy_v, [rows]))

        inf16 = jnp.full((16,), jnp.inf, jnp.float32)
        zero16 = jnp.zeros((16,), jnp.int32)
        best = [[inf16 for _ in range(NS)] for _ in range(NH)]
        bidx = [[zero16 for _ in range(NS)] for _ in range(NH)]
        for s in range(NS):
            for gi in range(GB):
                g = s * GB + gi
                gx = gcx_ch[g // 16][g % 16]
                gy = gcy_ch[g // 16][g % 16]
                for h in range(NH):
                    dx = cx_h[h] - gx
                    dy = cy_h[h] - gy
                    d2 = dx * dx + dy * dy
                    m = d2 < best[h][s]
                    best[h][s] = jnp.where(m, d2, best[h][s])
                    bidx[h][s] = jnp.where(m, jnp.int32(g), bidx[h][s])

        for h in range(NH):
            b, bi = best[h][0], bidx[h][0]
            for s in range(1, NS):
                m = best[h][s] < b
                b = jnp.where(m, best[h][s], b)
                bi = jnp.where(m, bidx[h][s], bi)

            glab = plsc.load_gather(glb_v, [bi])

            plsc.store_scatter(bidx_v, [rows_h[h]], bi)
            plsc.store_scatter(d2_v, [rows_h[h]], b)
            plsc.store_scatter(glbo_v, [rows_h[h]], glab.astype(jnp.int32))
        return carry

    lax.fori_loop(0, CHUNKS // NH, chunk, 0)

    pltpu.sync_copy(bidx_v, bidx_hbm.at[pl.ds(base, RPW)])
    pltpu.sync_copy(d2_v, d2_hbm.at[pl.ds(base, RPW)])
    pltpu.sync_copy(glbo_v, glbo_hbm.at[pl.ds(base, RPW)])


@jax.jit
def _run(reg_pred, targets, cls_pred):
    nb = NP // TB
    pad = NP - N
    rx0 = jnp.pad(reg_pred[:, 0], (0, pad))
    ry0 = jnp.pad(reg_pred[:, 1], (0, pad))
    rx1 = jnp.pad(reg_pred[:, 2], (0, pad))
    ry1 = jnp.pad(reg_pred[:, 3], (0, pad))
    cx, cy, gcx, gcy, glb = pl.pallas_call(
        _tc1_body,
        grid=(nb,),
        in_specs=[
            pl.BlockSpec((TB,), lambda i: (i,)),
            pl.BlockSpec((TB,), lambda i: (i,)),
            pl.BlockSpec((TB,), lambda i: (i,)),
            pl.BlockSpec((TB,), lambda i: (i,)),
            pl.BlockSpec((G, 5), lambda i: (0, 0)),
        ],
        out_specs=(
            pl.BlockSpec((TB,), lambda i: (i,)),
            pl.BlockSpec((TB,), lambda i: (i,)),
            pl.BlockSpec((G,), lambda i: (0,)),
            pl.BlockSpec((G,), lambda i: (0,)),
            pl.BlockSpec((G,), lambda i: (0,)),
        ),
        out_shape=(
            jax.ShapeDtypeStruct((NP,), jnp.float32),
            jax.ShapeDtypeStruct((NP,), jnp.float32),
            jax.ShapeDtypeStruct((G,), jnp.float32),
            jax.ShapeDtypeStruct((G,), jnp.float32),
            jax.ShapeDtypeStruct((G,), jnp.float32),
        ),
    )(rx0, ry0, rx1, ry1, targets)

    mesh = plsc.VectorSubcoreMesh(core_axis_name="c", subcore_axis_name="s")
    sc = pl.kernel(
        _sc_body,
        mesh=mesh,
        compiler_params=pltpu.CompilerParams(needs_layout_passes=False),
        out_type=(
            jax.ShapeDtypeStruct((N,), jnp.int32),
            jax.ShapeDtypeStruct((N,), jnp.float32),
            jax.ShapeDtypeStruct((N,), jnp.int32),
        ),
        scratch_types=[
            pltpu.VMEM((RPW,), jnp.float32),
            pltpu.VMEM((RPW,), jnp.float32),
            pltpu.VMEM((G,), jnp.float32),
            pltpu.VMEM((G,), jnp.float32),
            pltpu.VMEM((G,), jnp.float32),
            pltpu.VMEM((RPW,), jnp.int32),
            pltpu.VMEM((RPW,), jnp.float32),
            pltpu.VMEM((RPW,), jnp.int32),
            pltpu.SemaphoreType.DMA,
        ],
    )
    bidx, d2m, glbi = sc(cx, cy, gcx, gcy, glb)

    sidx = pl.pallas_call(
        _tc2_body,
        grid=(nb,),
        in_specs=[pl.BlockSpec((TB, C), lambda i: (i, 0))],
        out_specs=pl.BlockSpec((TB,), lambda i: (i,)),
        out_shape=jax.ShapeDtypeStruct((NP,), jnp.int32),
        compiler_params=pltpu.CompilerParams(vmem_limit_bytes=6 * 1024 * 1024),
    )(cls_pred)

    asg, dis, lbl = pl.pallas_call(
        _tc3_body,
        out_shape=(
            jax.ShapeDtypeStruct((N,), jnp.int32),
            jax.ShapeDtypeStruct((N,), jnp.float32),
            jax.ShapeDtypeStruct((N,), jnp.int32),
        ),
    )(bidx, d2m, glbi, sidx[:N])
    return asg, dis, lbl


def kernel(reg_pred, targets, num_level_bboxes, cls_pred):
    asg, dis, lbl = _run(reg_pred, targets, cls_pred)
    return (asg, dis, lbl, reg_pred, targets)


# speedup vs baseline: 1.5268x; 1.1548x over previous
import jax
import jax.numpy as jnp
from jax import lax
from jax.experimental import pallas as pl
from jax.experimental.pallas import tpu as pltpu
from jax.experimental.pallas import tpu_sc as plsc

N = 20000
G = 128
C = 80
NP = 20480
TB = 2048
NW = 32
RPW = 640
CHUNKS = RPW // 16


def _tc1_body(rx0_ref, ry0_ref, rx1_ref, ry1_ref, tgt_ref,
              cx_ref, cy_ref, gcx_ref, gcy_ref, glb_ref):
    tgtt = jnp.transpose(tgt_ref[...])
    cx_ref[pl.ds(0, N)] = (rx0_ref[...] + rx1_ref[...]) / 2.0
    cy_ref[pl.ds(0, N)] = (ry0_ref[...] + ry1_ref[...]) / 2.0
    gcx_ref[...] = (tgtt[0] + tgtt[2]) / 2.0
    gcy_ref[...] = (tgtt[1] + tgtt[3]) / 2.0
    glb_ref[...] = tgtt[4]


def _tc2_body(cls_ref, sidx_ref):
    clst = jnp.transpose(cls_ref[...])
    maxv = jnp.max(clst, axis=0)
    ciota = lax.broadcasted_iota(jnp.int32, clst.shape, 0)
    cidx = jnp.min(jnp.where(clst == maxv[None, :], ciota, C), axis=0)
    sidx_ref[...] = jnp.where(maxv > 0.0, cidx, -1)


def _tc3_body(bidx_ref, d2_ref, glb_ref, sidx_ref, asg_ref, dis_ref, lbl_ref):
    bi = bidx_ref[...]
    glab_i = glb_ref[...]
    pos = sidx_ref[...] == glab_i
    asg_ref[...] = jnp.where(pos, bi + 1, 0)
    dis_ref[...] = jnp.sqrt(d2_ref[...])
    lbl_ref[...] = jnp.where(pos, glab_i, -1)


def _sc_body(cx_hbm, cy_hbm, gcx_hbm, gcy_hbm, glb_hbm,
             bidx_hbm, d2_hbm, glbo_hbm,
             cx_v, cy_v, gcx_v, gcy_v, glb_v,
             bidx_v, d2_v, glbo_v, sem):
    wid = lax.axis_index("s") * 2 + lax.axis_index("c")
    base = jnp.minimum(wid * RPW, N - RPW)

    iota = jnp.arange(16, dtype=jnp.int32)

    cps = [
        pltpu.async_copy(cx_hbm.at[pl.ds(base, RPW)], cx_v, sem),
        pltpu.async_copy(cy_hbm.at[pl.ds(base, RPW)], cy_v, sem),
        pltpu.async_copy(gcx_hbm, gcx_v, sem),
        pltpu.async_copy(gcy_hbm, gcy_v, sem),
        pltpu.async_copy(glb_hbm, glb_v, sem),
    ]
    for cp in cps:
        cp.wait()

    gcx_ch = [gcx_v[pl.ds(16 * k, 16)] for k in range(G // 16)]
    gcy_ch = [gcy_v[pl.ds(16 * k, 16)] for k in range(G // 16)]

    NH = 2
    NS = 4
    GB = G // NS

    def chunk(j, carry):
        rows_h, cx_h, cy_h = [], [], []
        for h in range(NH):
            rows = iota + (j * (16 * NH) + 16 * h)
            rows_h.append(rows)
            cx_h.append(plsc.load_gather(cx_v, [rows]))
            cy_h.append(plsc.load_gather(cy_v, [rows]))

        inf16 = jnp.full((16,), jnp.inf, jnp.float32)
        zero16 = jnp.zeros((16,), jnp.int32)
        best = [[inf16 for _ in range(NS)] for _ in range(NH)]
        bidx = [[zero16 for _ in range(NS)] for _ in range(NH)]
        for s in range(NS):
            for gi in range(GB):
                g = s * GB + gi
                gx = gcx_ch[g // 16][g % 16]
                gy = gcy_ch[g // 16][g % 16]
                for h in range(NH):
                    dx = cx_h[h] - gx
                    dy = cy_h[h] - gy
                    d2 = dx * dx + dy * dy
                    m = d2 < best[h][s]
                    best[h][s] = jnp.where(m, d2, best[h][s])
                    bidx[h][s] = jnp.where(m, jnp.int32(g), bidx[h][s])

        for h in range(NH):
            b, bi = best[h][0], bidx[h][0]
            for s in range(1, NS):
                m = best[h][s] < b
                b = jnp.where(m, best[h][s], b)
                bi = jnp.where(m, bidx[h][s], bi)

            glab = plsc.load_gather(glb_v, [bi])

            plsc.store_scatter(bidx_v, [rows_h[h]], bi)
            plsc.store_scatter(d2_v, [rows_h[h]], b)
            plsc.store_scatter(glbo_v, [rows_h[h]], glab.astype(jnp.int32))
        return carry

    lax.fori_loop(0, CHUNKS // NH, chunk, 0)

    pltpu.sync_copy(bidx_v, bidx_hbm.at[pl.ds(base, RPW)])
    pltpu.sync_copy(d2_v, d2_hbm.at[pl.ds(base, RPW)])
    pltpu.sync_copy(glbo_v, glbo_hbm.at[pl.ds(base, RPW)])


@jax.jit
def _run(reg_pred, targets, cls_pred):
    nb = NP // TB
    rx0 = reg_pred[:, 0]
    ry0 = reg_pred[:, 1]
    rx1 = reg_pred[:, 2]
    ry1 = reg_pred[:, 3]
    cx, cy, gcx, gcy, glb = pl.pallas_call(
        _tc1_body,
        out_shape=(
            jax.ShapeDtypeStruct((NP,), jnp.float32),
            jax.ShapeDtypeStruct((NP,), jnp.float32),
            jax.ShapeDtypeStruct((G,), jnp.float32),
            jax.ShapeDtypeStruct((G,), jnp.float32),
            jax.ShapeDtypeStruct((G,), jnp.float32),
        ),
    )(rx0, ry0, rx1, ry1, targets)

    mesh = plsc.VectorSubcoreMesh(core_axis_name="c", subcore_axis_name="s")
    sc = pl.kernel(
        _sc_body,
        mesh=mesh,
        compiler_params=pltpu.CompilerParams(needs_layout_passes=False),
        out_type=(
            jax.ShapeDtypeStruct((N,), jnp.int32),
            jax.ShapeDtypeStruct((N,), jnp.float32),
            jax.ShapeDtypeStruct((N,), jnp.int32),
        ),
        scratch_types=[
            pltpu.VMEM((RPW,), jnp.float32),
            pltpu.VMEM((RPW,), jnp.float32),
            pltpu.VMEM((G,), jnp.float32),
            pltpu.VMEM((G,), jnp.float32),
            pltpu.VMEM((G,), jnp.float32),
            pltpu.VMEM((RPW,), jnp.int32),
            pltpu.VMEM((RPW,), jnp.float32),
            pltpu.VMEM((RPW,), jnp.int32),
            pltpu.SemaphoreType.DMA,
        ],
    )
    bidx, d2m, glbi = sc(cx, cy, gcx, gcy, glb)

    sidx = pl.pallas_call(
        _tc2_body,
        grid=(nb,),
        in_specs=[pl.BlockSpec((TB, C), lambda i: (i, 0))],
        out_specs=pl.BlockSpec((TB,), lambda i: (i,)),
        out_shape=jax.ShapeDtypeStruct((NP,), jnp.int32),
        compiler_params=pltpu.CompilerParams(vmem_limit_bytes=6 * 1024 * 1024),
    )(cls_pred)

    asg, dis, lbl = pl.pallas_call(
        _tc3_body,
        out_shape=(
            jax.ShapeDtypeStruct((N,), jnp.int32),
            jax.ShapeDtypeStruct((N,), jnp.float32),
            jax.ShapeDtypeStruct((N,), jnp.int32),
        ),
    )(bidx, d2m, glbi, sidx[:N])
    return asg, dis, lbl


def kernel(reg_pred, targets, num_level_bboxes, cls_pred):
    asg, dis, lbl = _run(reg_pred, targets, cls_pred)
    return (asg, dis, lbl, reg_pred, targets)
